# Initial kernel scaffold; baseline (speedup 1.0000x reference)
import jax
import jax.numpy as jnp
from jax.experimental import pallas as pl


def kernel(x, Wq, bq, Wk, bk, Wv, bv, cw1, cb1, cw2, cb2, k):
    raise NotImplementedError("write your pallas kernel here")


# speedup vs baseline: 13.4342x; 13.4342x over previous
import functools
import math

import jax
import jax.numpy as jnp
from jax.experimental import pallas as pl
from jax.experimental.pallas import tpu as pltpu

N = 8192
D = 768
HEADS = 12
DS = D // HEADS
HID = 24
BN = 1024
NB = N // BN


def _mhap_kernel(x_ref, cw1_ref, cw2_ref, wq_ref, wk_ref, wv_ref,
                 bq_ref, bk_ref, bv_ref, cb1_ref, cb2_ref,
                 out_ref,
                 xc_acc, rs1_acc, ws_s, cbias_s, m_s, s_s, u_s):
    p = pl.program_id(0)
    i = pl.program_id(1)

    @pl.when(jnp.logical_and(p == 0, i == 0))
    def _init0():
        xc_acc[...] = jnp.zeros_like(xc_acc)
        rs1_acc[...] = jnp.zeros_like(rs1_acc)

    @pl.when(p == 0)
    def _pass0():
        cw1_blk = cw1_ref[...]
        xc_acc[...] += jax.lax.dot_general(
            cw1_blk, x_ref[...], (((1,), (0,)), ((), ())),
            preferred_element_type=jnp.float32)
        rs1_acc[...] += jnp.sum(cw1_blk, axis=1, keepdims=True)

    @pl.when(jnp.logical_and(p == 1, i == 0))
    def _prelude():
        h = jax.lax.dot_general(
            xc_acc[...], wq_ref[...], (((1,), (1,)), ((), ())),
            preferred_element_type=jnp.float32)
        h = h + rs1_acc[...] * bq_ref[...] + cb1_ref[...]
        h = jnp.maximum(h, 0.0)
        q = jax.lax.dot_general(
            cw2_ref[...], h, (((1,), (0,)), ((), ())),
            preferred_element_type=jnp.float32) + cb2_ref[...]
        hh = jax.lax.broadcasted_iota(jnp.int32, (HEADS, D), 0)
        dd = jax.lax.broadcasted_iota(jnp.int32, (HEADS, D), 1)
        qh = jnp.where(dd // DS == hh, q, 0.0)
        inv = 1.0 / math.sqrt(DS)
        ws_s[...] = inv * jax.lax.dot_general(
            qh, wk_ref[...], (((1,), (0,)), ((), ())),
            preferred_element_type=jnp.float32)
        cbias_s[...] = inv * jnp.sum(qh * bk_ref[...], axis=1, keepdims=True)
        m_s[...] = jnp.full_like(m_s, -1e30)
        s_s[...] = jnp.zeros_like(s_s)
        u_s[...] = jnp.zeros_like(u_s)

    @pl.when(p == 1)
    def _pass1():
        xb = x_ref[...]
        sb = jax.lax.dot_general(
            ws_s[...], xb, (((1,), (1,)), ((), ())),
            preferred_element_type=jnp.float32) + cbias_s[...]
        bm = jnp.max(sb, axis=1, keepdims=True)
        new_m = jnp.maximum(m_s[...], bm)
        alpha = jnp.exp(m_s[...] - new_m)
        e = jnp.exp(sb - new_m)
        s_s[...] = s_s[...] * alpha + jnp.sum(e, axis=1, keepdims=True)
        u_s[...] = u_s[...] * alpha + jax.lax.dot_general(
            e, xb, (((1,), (0,)), ((), ())),
            preferred_element_type=jnp.float32)
        m_s[...] = new_m

    @pl.when(jnp.logical_and(p == 1, i == NB - 1))
    def _epilogue():
        un = u_s[...] / s_s[...]
        of = jax.lax.dot_general(
            wv_ref[...], un, (((1,), (1,)), ((), ())),
            preferred_element_type=jnp.float32)
        dd = jax.lax.broadcasted_iota(jnp.int32, (D, HEADS), 0)
        hh = jax.lax.broadcasted_iota(jnp.int32, (D, HEADS), 1)
        sel = jnp.where(dd // DS == hh, of, 0.0)
        out_ref[...] = jnp.sum(sel, axis=1, keepdims=True) + bv_ref[...]


@functools.partial(jax.jit, static_argnames=())
def kernel(x, Wq, bq, Wk, bk, Wv, bv, cw1, cb1, cw2, cb2, k):
    del k
    x2 = x[0]
    cw1_2 = cw1[:, :, 0]
    cw2_2 = cw2[:, :, 0]
    bq2 = bq.reshape(1, D)
    bk2 = bk.reshape(1, D)
    bv2 = bv.reshape(D, 1)
    cb1_2 = cb1.reshape(HID, 1)
    cb2_2 = cb2.reshape(1, 1)

    out = pl.pallas_call(
        _mhap_kernel,
        grid=(2, NB),
        in_specs=[
            pl.BlockSpec((BN, D), lambda p, i: (i, 0)),
            pl.BlockSpec((HID, BN), lambda p, i: (0, i)),
            pl.BlockSpec((1, HID), lambda p, i: (0, 0)),
            pl.BlockSpec((D, D), lambda p, i: (0, 0)),
            pl.BlockSpec((D, D), lambda p, i: (0, 0)),
            pl.BlockSpec((D, D), lambda p, i: (0, 0)),
            pl.BlockSpec((1, D), lambda p, i: (0, 0)),
            pl.BlockSpec((1, D), lambda p, i: (0, 0)),
            pl.BlockSpec((D, 1), lambda p, i: (0, 0)),
            pl.BlockSpec((HID, 1), lambda p, i: (0, 0)),
            pl.BlockSpec((1, 1), lambda p, i: (0, 0)),
        ],
        out_specs=pl.BlockSpec((D, 1), lambda p, i: (0, 0)),
        out_shape=jax.ShapeDtypeStruct((D, 1), jnp.float32),
        scratch_shapes=[
            pltpu.VMEM((HID, D), jnp.float32),
            pltpu.VMEM((HID, 1), jnp.float32),
            pltpu.VMEM((HEADS, D), jnp.float32),
            pltpu.VMEM((HEADS, 1), jnp.float32),
            pltpu.VMEM((HEADS, 1), jnp.float32),
            pltpu.VMEM((HEADS, 1), jnp.float32),
            pltpu.VMEM((HEADS, D), jnp.float32),
        ],
        compiler_params=pltpu.CompilerParams(
            dimension_semantics=("arbitrary", "arbitrary"),
        ),
    )(x2, cw1_2, cw2_2, Wq, Wk, Wv, bq2, bk2, bv2, cb1_2, cb2_2)

    return out.reshape(1, 1, D)


# speedup vs baseline: 15.9884x; 1.1901x over previous
import functools
import math

import jax
import jax.numpy as jnp
from jax.experimental import pallas as pl
from jax.experimental.pallas import tpu as pltpu

N = 8192
D = 768
HEADS = 12
DS = D // HEADS
HID = 24
BN = 2048
NB = N // BN


def _mhap_kernel(x_ref, cw1_ref, cw2_ref, wq_ref, wk_ref, wv_ref,
                 bq_ref, bk_ref, bv_ref, cb1_ref, cb2_ref,
                 out_ref,
                 xc_acc, rs1_acc, ws_s, cbias_s, m_s, s_s, u_s, x_cache):
    p = pl.program_id(0)
    i = pl.program_id(1)

    @pl.when(jnp.logical_and(p == 0, i == 0))
    def _init0():
        xc_acc[...] = jnp.zeros_like(xc_acc)
        rs1_acc[...] = jnp.zeros_like(rs1_acc)

    @pl.when(p == 0)
    def _pass0():
        cw1_blk = cw1_ref[...]
        xb = x_ref[...]
        x_cache[pl.ds(i * BN, BN), :] = xb
        xc_acc[...] += jax.lax.dot_general(
            cw1_blk, xb, (((1,), (0,)), ((), ())),
            preferred_element_type=jnp.float32)
        rs1_acc[...] += jnp.sum(cw1_blk, axis=1, keepdims=True)

    @pl.when(jnp.logical_and(p == 1, i == 0))
    def _prelude():
        h = jax.lax.dot_general(
            xc_acc[...], wq_ref[...], (((1,), (1,)), ((), ())),
            preferred_element_type=jnp.float32)
        h = h + rs1_acc[...] * bq_ref[...] + cb1_ref[...]
        h = jnp.maximum(h, 0.0)
        q = jax.lax.dot_general(
            cw2_ref[...], h, (((1,), (0,)), ((), ())),
            preferred_element_type=jnp.float32) + cb2_ref[...]
        hh = jax.lax.broadcasted_iota(jnp.int32, (HEADS, D), 0)
        dd = jax.lax.broadcasted_iota(jnp.int32, (HEADS, D), 1)
        qh = jnp.where(dd // DS == hh, q, 0.0)
        inv = 1.0 / math.sqrt(DS)
        ws_s[...] = inv * jax.lax.dot_general(
            qh, wk_ref[...], (((1,), (0,)), ((), ())),
            preferred_element_type=jnp.float32)
        cbias_s[...] = inv * jnp.sum(qh * bk_ref[...], axis=1, keepdims=True)
        m_s[...] = jnp.full_like(m_s, -1e30)
        s_s[...] = jnp.zeros_like(s_s)
        u_s[...] = jnp.zeros_like(u_s)

    @pl.when(p == 1)
    def _pass1():
        xb = x_cache[pl.ds(i * BN, BN), :]
        sb = jax.lax.dot_general(
            ws_s[...], xb, (((1,), (1,)), ((), ())),
            preferred_element_type=jnp.float32) + cbias_s[...]
        bm = jnp.max(sb, axis=1, keepdims=True)
        new_m = jnp.maximum(m_s[...], bm)
        alpha = jnp.exp(m_s[...] - new_m)
        e = jnp.exp(sb - new_m)
        s_s[...] = s_s[...] * alpha + jnp.sum(e, axis=1, keepdims=True)
        u_s[...] = u_s[...] * alpha + jax.lax.dot_general(
            e, xb, (((1,), (0,)), ((), ())),
            preferred_element_type=jnp.float32)
        m_s[...] = new_m

    @pl.when(jnp.logical_and(p == 1, i == NB - 1))
    def _epilogue():
        un = u_s[...] / s_s[...]
        of = jax.lax.dot_general(
            wv_ref[...], un, (((1,), (1,)), ((), ())),
            preferred_element_type=jnp.float32)
        dd = jax.lax.broadcasted_iota(jnp.int32, (D, HEADS), 0)
        hh = jax.lax.broadcasted_iota(jnp.int32, (D, HEADS), 1)
        sel = jnp.where(dd // DS == hh, of, 0.0)
        out_ref[...] = jnp.sum(sel, axis=1, keepdims=True) + bv_ref[...]


@functools.partial(jax.jit, static_argnames=())
def kernel(x, Wq, bq, Wk, bk, Wv, bv, cw1, cb1, cw2, cb2, k):
    del k
    x2 = x[0]
    cw1_2 = cw1[:, :, 0]
    cw2_2 = cw2[:, :, 0]
    bq2 = bq.reshape(1, D)
    bk2 = bk.reshape(1, D)
    bv2 = bv.reshape(D, 1)
    cb1_2 = cb1.reshape(HID, 1)
    cb2_2 = cb2.reshape(1, 1)

    out = pl.pallas_call(
        _mhap_kernel,
        grid=(2, NB),
        in_specs=[
            pl.BlockSpec((BN, D),
                         lambda p, i: (jnp.where(p == 0, i, NB - 1), 0)),
            pl.BlockSpec((HID, BN), lambda p, i: (0, i)),
            pl.BlockSpec((1, HID), lambda p, i: (0, 0)),
            pl.BlockSpec((D, D), lambda p, i: (0, 0)),
            pl.BlockSpec((D, D), lambda p, i: (0, 0)),
            pl.BlockSpec((D, D), lambda p, i: (0, 0)),
            pl.BlockSpec((1, D), lambda p, i: (0, 0)),
            pl.BlockSpec((1, D), lambda p, i: (0, 0)),
            pl.BlockSpec((D, 1), lambda p, i: (0, 0)),
            pl.BlockSpec((HID, 1), lambda p, i: (0, 0)),
            pl.BlockSpec((1, 1), lambda p, i: (0, 0)),
        ],
        out_specs=pl.BlockSpec((D, 1), lambda p, i: (0, 0)),
        out_shape=jax.ShapeDtypeStruct((D, 1), jnp.float32),
        scratch_shapes=[
            pltpu.VMEM((HID, D), jnp.float32),
            pltpu.VMEM((HID, 1), jnp.float32),
            pltpu.VMEM((HEADS, D), jnp.float32),
            pltpu.VMEM((HEADS, 1), jnp.float32),
            pltpu.VMEM((HEADS, 1), jnp.float32),
            pltpu.VMEM((HEADS, 1), jnp.float32),
            pltpu.VMEM((HEADS, D), jnp.float32),
            pltpu.VMEM((N, D), jnp.float32),
        ],
        compiler_params=pltpu.CompilerParams(
            dimension_semantics=("arbitrary", "arbitrary"),
        ),
    )(x2, cw1_2, cw2_2, Wq, Wk, Wv, bq2, bk2, bv2, cb1_2, cb2_2)

    return out.reshape(1, 1, D)
